# Initial kernel scaffold; baseline (speedup 1.0000x reference)
import jax
import jax.numpy as jnp
from jax.experimental import pallas as pl


def kernel(gid, senders, receivers, is_training, node_embeddings, W1, b1, W2, b2):
    raise NotImplementedError("write your pallas kernel here")


# speedup vs baseline: 6.2254x; 6.2254x over previous
import functools

import jax
import jax.numpy as jnp
from jax import lax
from jax.experimental import pallas as pl
from jax.experimental.pallas import tpu as pltpu
from jax.experimental.pallas import tpu_sc as plsc

N = 10000
D = 128
E = 320000
NC = 2
NS = 16
NW = NC * NS
CH = 128
CPT = -(-E // (NW * CH))
EP = NW * CPT * CH
NB = 10240
RPT = NB // NS

_mesh = plsc.VectorSubcoreMesh(
    core_axis_name="c", subcore_axis_name="s", num_cores=NC, num_subcores=NS
)


def _deg_body(sd_hbm, rr_hbm, z_hbm, o_hbm, outs_hbm, outr_hbm, acc_s, acc_r, sbuf, rbuf, obuf):
    c = lax.axis_index("c")
    s = lax.axis_index("s")
    wid = s * NC + c
    pltpu.sync_copy(z_hbm, acc_s.at[pl.ds(s * RPT, RPT)])
    pltpu.sync_copy(z_hbm, acc_r.at[pl.ds(s * RPT, RPT)])
    pltpu.sync_copy(o_hbm, obuf)
    pltpu.sync_copy(sd_hbm.at[wid], sbuf)
    pltpu.sync_copy(rr_hbm.at[wid], rbuf)
    plsc.subcore_barrier()

    def it(j, carry):
        pltpu.sync_copy(obuf, acc_s.at[sbuf.at[j]], add=True)
        pltpu.sync_copy(obuf, acc_r.at[rbuf.at[j]], add=True)
        return carry

    lax.fori_loop(0, CPT, it, 0)
    plsc.subcore_barrier()
    pltpu.sync_copy(acc_s.at[pl.ds(s * RPT, RPT)], outs_hbm.at[c, 0, pl.ds(s * RPT, RPT)])
    pltpu.sync_copy(acc_r.at[pl.ds(s * RPT, RPT)], outr_hbm.at[c, 0, pl.ds(s * RPT, RPT)])


_deg_call = pl.kernel(
    _deg_body,
    out_type=(jax.ShapeDtypeStruct((NC, 1, NB), jnp.float32),
              jax.ShapeDtypeStruct((NC, 1, NB), jnp.float32)),
    mesh=_mesh,
    scratch_types=[
        pltpu.VMEM_SHARED((NB,), jnp.float32),
        pltpu.VMEM_SHARED((NB,), jnp.float32),
        pltpu.VMEM((CPT, CH), jnp.int32),
        pltpu.VMEM((CPT, CH), jnp.int32),
        pltpu.VMEM((CH,), jnp.float32),
    ],
)


def _scat_body(xn_hbm, sg_hbm, rr_hbm, zr_hbm, out_hbm, acc, sbuf, rbuf, rows):
    c = lax.axis_index("c")
    s = lax.axis_index("s")
    wid = s * NC + c
    pltpu.sync_copy(zr_hbm, acc.at[pl.ds(s * RPT, RPT)])
    pltpu.sync_copy(sg_hbm.at[wid], sbuf)
    pltpu.sync_copy(rr_hbm.at[wid], rbuf)
    plsc.subcore_barrier()

    def it(j, carry):
        pltpu.sync_copy(xn_hbm.at[sbuf.at[j]], rows)
        pltpu.sync_copy(rows, acc.at[rbuf.at[j]], add=True)
        return carry

    lax.fori_loop(0, CPT, it, 0)
    plsc.subcore_barrier()
    pltpu.sync_copy(acc.at[pl.ds(s * RPT, RPT)], out_hbm.at[c, pl.ds(s * RPT, RPT)])


_scat_call = pl.kernel(
    _scat_body,
    out_type=jax.ShapeDtypeStruct((NC, NB, D), jnp.float32),
    mesh=_mesh,
    scratch_types=[
        pltpu.VMEM_SHARED((NB, D), jnp.float32),
        pltpu.VMEM((CPT, CH), jnp.int32),
        pltpu.VMEM((CPT, CH), jnp.int32),
        pltpu.VMEM((CH, D), jnp.float32),
    ],
)

BN = 2000


def _tc_body(do_relu, x, y0, y1, xn, srb, wt, wb, b, out):
    z = (y0[...] + y1[...] + xn[...]) * srb[...]
    o = (
        jnp.dot(x[...], wt[...], precision=lax.Precision.HIGHEST,
                preferred_element_type=jnp.float32)
        + jnp.dot(z, wb[...], precision=lax.Precision.HIGHEST,
                  preferred_element_type=jnp.float32)
        + b[...]
    )
    if do_relu:
        o = jnp.maximum(o, 0.0)
    out[...] = o


def _tc_layer(do_relu):
    row = pl.BlockSpec((BN, D), lambda i: (i, 0))
    full = pl.BlockSpec((D, D), lambda i: (0, 0))
    brow = pl.BlockSpec((1, D), lambda i: (0, 0))
    return pl.pallas_call(
        functools.partial(_tc_body, do_relu),
        grid=(N // BN,),
        in_specs=[row, row, row, row, row, full, full, brow],
        out_specs=row,
        out_shape=jax.ShapeDtypeStruct((N, D), jnp.float32),
    )


_tc1 = _tc_layer(True)
_tc2 = _tc_layer(False)


def kernel(gid, senders, receivers, is_training, node_embeddings, W1, b1, W2, b2):
    del is_training
    x = jnp.take(node_embeddings, gid, axis=0)

    pad = EP - E
    sg = jnp.concatenate([senders, jnp.zeros((pad,), jnp.int32)]).reshape(NW, CPT, CH)
    sd = jnp.concatenate([senders, jnp.full((pad,), N, jnp.int32)]).reshape(NW, CPT, CH)
    rr = jnp.concatenate([receivers, jnp.full((pad,), N, jnp.int32)]).reshape(NW, CPT, CH)
    zvec = jnp.zeros((RPT,), jnp.float32)
    ones = jnp.ones((CH,), jnp.float32)
    zrows = jnp.zeros((RPT, D), jnp.float32)

    dS, dR = _deg_call(sd, rr, zvec, ones)
    deg_s = 1.0 + dS[0, 0, :N] + dS[1, 0, :N]
    cnt_r = 1.0 + dR[0, 0, :N] + dR[1, 0, :N]
    ss = lax.rsqrt(deg_s)[:, None]
    srb = jnp.broadcast_to((cnt_r * jnp.sqrt(cnt_r))[:, None] ** -1.0, (N, D))

    xn1 = x * ss
    y1p = _scat_call(xn1, sg, rr, zrows)
    h1 = _tc1(x, y1p[0, :N], y1p[1, :N], xn1, srb, W1[:D], W1[D:], b1.reshape(1, D))

    xn2 = h1 * ss
    y2p = _scat_call(xn2, sg, rr, zrows)
    return _tc2(h1, y2p[0, :N], y2p[1, :N], xn2, srb, W2[:D], W2[D:], b2.reshape(1, D))
